# baseline (speedup 1.0000x reference)
import functools

import jax
import jax.numpy as jnp
from jax import lax
from jax.experimental import pallas as pl
from jax.experimental.pallas import tpu as pltpu
from jax.experimental.pallas import tpu_sc as plsc

N_EVENT = 5000
N_IOC = 5000
N = N_EVENT + N_IOC
H = 128
NB = 8
R = 3
NC = 2
NS = 16
LANE = 128

ACCA_ROWS = 10112
DUMMYA = 10100
LANEA = 64
STEPSA = 98
ACCB_ROWS = 5120
DUMMYB = 5100
LANEB = 64
STEPSB = 60
ACCC_ROWS = 15104
DUMMYC = 15100
LANEC = 128
STEPSC = 79
STEPS2 = 50


def _proj_body(x_ref, w_ref, b_ref, g_ref, bt_ref, o_ref):
    y = jnp.dot(x_ref[...], w_ref[...], preferred_element_type=jnp.float32)
    y = y + b_ref[...]
    mu = jnp.mean(y, axis=-1, keepdims=True)
    var = jnp.mean((y - mu) ** 2, axis=-1, keepdims=True)
    y = (y - mu) * lax.rsqrt(var + 1e-5) * g_ref[...] + bt_ref[...]
    o_ref[...] = jnp.maximum(y, 0.0)


def _project(x, w, b, g, bt, blk=1000):
    n, d = x.shape
    return pl.pallas_call(
        _proj_body,
        grid=(n // blk,),
        in_specs=[
            pl.BlockSpec((blk, d), lambda i: (i, 0)),
            pl.BlockSpec((d, H), lambda i: (0, 0)),
            pl.BlockSpec((1, H), lambda i: (0, 0)),
            pl.BlockSpec((1, H), lambda i: (0, 0)),
            pl.BlockSpec((1, H), lambda i: (0, 0)),
        ],
        out_specs=pl.BlockSpec((blk, H), lambda i: (i, 0)),
        out_shape=jax.ShapeDtypeStruct((n, H), jnp.float32),
    )(x, w, b.reshape(1, H), g.reshape(1, H), bt.reshape(1, H))


def _relw_body(bases_ref, comp_ref, o_ref):
    l = pl.program_id(0)
    r = pl.program_id(1)
    acc = comp_ref[l, r, 0] * bases_ref[0, 0]
    for b in range(1, NB):
        acc = acc + comp_ref[l, r, b] * bases_ref[0, b]
    o_ref[0, 0] = acc


def _relation_weights(bases, comp):
    L = bases.shape[0]
    return pl.pallas_call(
        _relw_body,
        grid=(L, R),
        in_specs=[
            pl.BlockSpec((1, NB, H, H), lambda l, r: (l, 0, 0, 0)),
            pl.BlockSpec(memory_space=pltpu.SMEM),
        ],
        out_specs=pl.BlockSpec((1, 1, H, H), lambda l, r: (l, r, 0, 0)),
        out_shape=jax.ShapeDtypeStruct((L, R, H, H), jnp.float32),
    )(bases, comp)


def _make_update_body(nrel):
    def body(*refs):
        h_ref = refs[0]
        p_refs = refs[1:1 + nrel]
        c_refs = refs[1 + nrel:1 + 2 * nrel]
        w_refs = refs[1 + 2 * nrel:1 + 3 * nrel]
        rw_ref, rb_ref, g_ref, bt_ref, o_ref = refs[1 + 3 * nrel:]
        h = h_ref[...]
        out = jnp.dot(h, rw_ref[...], preferred_element_type=jnp.float32)
        out = out + rb_ref[...]
        for i in range(nrel):
            p = p_refs[i][...]
            a = p[0] + p[1]
            c = c_refs[i][...]
            cnt = c[0, :, 0:1] + c[1, :, 0:1]
            a = a * (1.0 / jnp.maximum(cnt, 1.0))
            out = out + jnp.dot(a, w_refs[i][...],
                                preferred_element_type=jnp.float32)
        mu = jnp.mean(out, axis=-1, keepdims=True)
        var = jnp.mean((out - mu) ** 2, axis=-1, keepdims=True)
        out = (out - mu) * lax.rsqrt(var + 1e-5) * g_ref[...] + bt_ref[...]
        o_ref[...] = jnp.maximum(out, 0.0) + h
    return body


def _update(h, h_row_blk, parts, cnts, p_offs, c_offs, ws, rw, rb, g, bt,
            blk=1000):
    nrel = len(p_offs)
    in_specs = [pl.BlockSpec((blk, H), lambda i, o=h_row_blk: (o + i, 0))]
    args = [h]
    for o in p_offs:
        in_specs.append(pl.BlockSpec((2, blk, H), lambda i, o=o: (0, o + i, 0)))
        args.append(parts)
    for o in c_offs:
        in_specs.append(pl.BlockSpec((2, blk, H), lambda i, o=o: (0, o + i, 0)))
        args.append(cnts)
    for w in ws:
        in_specs.append(pl.BlockSpec((H, H), lambda i: (0, 0)))
        args.append(w)
    for a in (rw, rb.reshape(1, H), g.reshape(1, H), bt.reshape(1, H)):
        in_specs.append(pl.BlockSpec(a.shape, lambda i: (0,) * a.ndim))
        args.append(a)
    return pl.pallas_call(
        _make_update_body(nrel),
        grid=(N_EVENT // blk,),
        in_specs=in_specs,
        out_specs=pl.BlockSpec((blk, H), lambda i: (i, 0)),
        out_shape=jax.ShapeDtypeStruct((N_EVENT, H), jnp.float32),
    )(*args)


def _make_sc_agg(n_steps, lane, acc_rows):
    mesh = plsc.VectorSubcoreMesh(core_axis_name="c", subcore_axis_name="s",
                                  num_cores=NC, num_subcores=NS)
    per_tile = acc_rows // NS

    @functools.partial(
        pl.kernel,
        out_type=jax.ShapeDtypeStruct((NC * acc_rows, H), jnp.float32),
        mesh=mesh,
        scratch_types=[
            pltpu.VMEM((n_steps, lane), jnp.int32),
            pltpu.VMEM((n_steps, lane), jnp.int32),
            pltpu.VMEM((lane, H), jnp.float32),
            pltpu.VMEM((lane, H), jnp.float32),
            pltpu.VMEM_SHARED((acc_rows, H), jnp.float32),
            pltpu.SemaphoreType.DMA,
            pltpu.SemaphoreType.DMA,
        ],
    )
    def agg(src_hbm, dst_hbm, table_hbm, zero_hbm, out_hbm,
            src_v, dst_v, buf0, buf1, acc, sem0, sem1):
        cidx = lax.axis_index("c")
        sidx = lax.axis_index("s")
        wid = cidx * NS + sidx
        pltpu.sync_copy(src_hbm.at[wid], src_v)
        pltpu.sync_copy(dst_hbm.at[wid], dst_v)
        base = sidx * per_tile
        pltpu.sync_copy(zero_hbm, acc.at[pl.ds(base, per_tile)])
        plsc.subcore_barrier()

        def pair(jj, carry):
            j0 = jj * 2
            d0 = pltpu.async_copy(table_hbm.at[src_v.at[j0]], buf0, sem0)
            d1 = pltpu.async_copy(table_hbm.at[src_v.at[j0 + 1]], buf1, sem1)
            d0.wait()
            pltpu.sync_copy(buf0, acc.at[dst_v.at[j0]], add=True)
            d1.wait()
            pltpu.sync_copy(buf1, acc.at[dst_v.at[j0 + 1]], add=True)
            return carry

        lax.fori_loop(0, n_steps // 2, pair, 0)
        plsc.subcore_barrier()
        pltpu.sync_copy(acc.at[pl.ds(base, per_tile)],
                        out_hbm.at[pl.ds(cidx * acc_rows + base, per_tile)])

    return agg


_make_sc_agg = functools.lru_cache(maxsize=None)(_make_sc_agg)


def _sc_aggA(*args):
    return _make_sc_agg(STEPSA, LANEA, ACCA_ROWS)(*args)


def _sc_aggB(*args):
    return _make_sc_agg(STEPSB, LANEB, ACCB_ROWS)(*args)


def _pad_edges(src, dst, n_steps, lane, dummy):
    tot = NC * NS * n_steps * lane
    pad = tot - dst.shape[0]
    dst = jnp.concatenate([dst, jnp.full((pad,), dummy, jnp.int32)])
    dst = dst.reshape(NC * NS, n_steps, lane)
    if src is None:
        return dst
    src = jnp.concatenate([src, jnp.zeros((pad,), jnp.int32)])
    return src.reshape(NC * NS, n_steps, lane), dst


def kernel(x_event, x_ioc, pW_e, pb_e, pg_e, pbt_e, pW_i, pb_i, pg_i, pbt_i,
           bases, comp, rootw, rootb, lng, lnb, ei0, ei1, ei2):
    h_e = _project(x_event, pW_e, pb_e, pg_e, pbt_e)
    h_i = _project(x_ioc, pW_i, pb_i, pg_i, pbt_i)
    h = jnp.concatenate([h_e, h_i], axis=0)

    sA = jnp.concatenate([ei1[0] + N_EVENT, ei2[0]])
    dA = jnp.concatenate([ei1[1], ei2[1] + N_EVENT])
    srcA, dstA = _pad_edges(sA, dA, STEPSA, LANEA, DUMMYA)
    srcB, dstB = _pad_edges(ei0[0], ei0[1], STEPSB, LANEB, DUMMYB)

    zerosA = jnp.zeros((ACCA_ROWS // NS, H), jnp.float32)
    zerosB = jnp.zeros((ACCB_ROWS // NS, H), jnp.float32)
    ones_tbl = jnp.ones((8, H), jnp.float32)

    cntA = _sc_aggA(jnp.zeros_like(srcA), dstA, ones_tbl,
                    zerosA).reshape(NC, ACCA_ROWS, H)
    cntB = _sc_aggB(jnp.zeros_like(srcB), dstB, ones_tbl,
                    zerosB).reshape(NC, ACCB_ROWS, H)
    w_rel = _relation_weights(bases, comp)

    pa1 = _sc_aggA(srcA, dstA, h, zerosA).reshape(NC, ACCA_ROWS, H)
    pb1 = _sc_aggB(srcB, dstB, h, zerosB).reshape(NC, ACCB_ROWS, H)
    h1_e = _update(h, 0, pa1, cntA, p_offs=(0, 5), c_offs=(0, 5),
                   ws=(w_rel[0, 1], w_rel[0, 2]), rw=rootw[0], rb=rootb[0],
                   g=lng[0], bt=lnb[0])
    h1_i = _update(h, 5, pb1, cntB, p_offs=(0,), c_offs=(0,),
                   ws=(w_rel[0, 0],), rw=rootw[0], rb=rootb[0],
                   g=lng[0], bt=lnb[0])
    h1 = jnp.concatenate([h1_e, h1_i], axis=0)

    pa2 = _sc_aggA(srcA, dstA, h1, zerosA).reshape(NC, ACCA_ROWS, H)
    out = _update(h1, 0, pa2, cntA, p_offs=(0, 5), c_offs=(0, 5),
                  ws=(w_rel[1, 1], w_rel[1, 2]), rw=rootw[1], rb=rootb[1],
                  g=lng[1], bt=lnb[1])
    return out


# speedup vs baseline: 30.5286x; 30.5286x over previous
import functools

import jax
import jax.numpy as jnp
from jax import lax
from jax.experimental import pallas as pl
from jax.experimental.pallas import tpu as pltpu
from jax.experimental.pallas import tpu_sc as plsc

N_EVENT = 5000
N_IOC = 5000
N = N_EVENT + N_IOC
H = 128
NB = 8
R = 3
NC = 2
NS = 16
LANE = 128

ACCA_ROWS = 10112
DUMMYA = 10016
LANEA = 64
STEPSA = 100
CHUNK = 20
ACCB_ROWS = 5120
DUMMYB = 5024
LANEB = 64
STEPSB = 60
ONES_ROWS = 1024
STEPS2 = 50


def _proj_body(x_ref, w_ref, b_ref, g_ref, bt_ref, o_ref):
    y = jnp.dot(x_ref[...], w_ref[...], preferred_element_type=jnp.float32)
    y = y + b_ref[...]
    mu = jnp.mean(y, axis=-1, keepdims=True)
    var = jnp.mean((y - mu) ** 2, axis=-1, keepdims=True)
    y = (y - mu) * lax.rsqrt(var + 1e-5) * g_ref[...] + bt_ref[...]
    o_ref[...] = jnp.maximum(y, 0.0)


def _project(x, w, b, g, bt, blk=1000):
    n, d = x.shape
    return pl.pallas_call(
        _proj_body,
        grid=(n // blk,),
        in_specs=[
            pl.BlockSpec((blk, d), lambda i: (i, 0)),
            pl.BlockSpec((d, H), lambda i: (0, 0)),
            pl.BlockSpec((1, H), lambda i: (0, 0)),
            pl.BlockSpec((1, H), lambda i: (0, 0)),
            pl.BlockSpec((1, H), lambda i: (0, 0)),
        ],
        out_specs=pl.BlockSpec((blk, H), lambda i: (i, 0)),
        out_shape=jax.ShapeDtypeStruct((n, H), jnp.float32),
    )(x, w, b.reshape(1, H), g.reshape(1, H), bt.reshape(1, H))


def _relw_body(bases_ref, comp_ref, o_ref):
    l = pl.program_id(0)
    r = pl.program_id(1)
    acc = comp_ref[l, r, 0] * bases_ref[0, 0]
    for b in range(1, NB):
        acc = acc + comp_ref[l, r, b] * bases_ref[0, b]
    o_ref[0, 0] = acc


def _relation_weights(bases, comp):
    L = bases.shape[0]
    return pl.pallas_call(
        _relw_body,
        grid=(L, R),
        in_specs=[
            pl.BlockSpec((1, NB, H, H), lambda l, r: (l, 0, 0, 0)),
            pl.BlockSpec(memory_space=pltpu.SMEM),
        ],
        out_specs=pl.BlockSpec((1, 1, H, H), lambda l, r: (l, r, 0, 0)),
        out_shape=jax.ShapeDtypeStruct((L, R, H, H), jnp.float32),
    )(bases, comp)


def _make_update_body(nrel):
    def body(*refs):
        h_ref = refs[0]
        p_refs = refs[1:1 + nrel]
        c_refs = refs[1 + nrel:1 + 2 * nrel]
        w_refs = refs[1 + 2 * nrel:1 + 3 * nrel]
        rw_ref, rb_ref, g_ref, bt_ref, o_ref = refs[1 + 3 * nrel:]
        h = h_ref[...]
        out = jnp.dot(h, rw_ref[...], preferred_element_type=jnp.float32)
        out = out + rb_ref[...]
        for i in range(nrel):
            p = p_refs[i][...]
            a = p[0] + p[1]
            c = c_refs[i][...]
            cnt = c[0, :, 0:1] + c[1, :, 0:1]
            a = a * (1.0 / jnp.maximum(cnt, 1.0))
            out = out + jnp.dot(a, w_refs[i][...],
                                preferred_element_type=jnp.float32)
        mu = jnp.mean(out, axis=-1, keepdims=True)
        var = jnp.mean((out - mu) ** 2, axis=-1, keepdims=True)
        out = (out - mu) * lax.rsqrt(var + 1e-5) * g_ref[...] + bt_ref[...]
        o_ref[...] = jnp.maximum(out, 0.0) + h
    return body


def _update(h, h_row_blk, parts, cnts, p_offs, c_offs, ws, rw, rb, g, bt,
            blk=1000):
    nrel = len(p_offs)
    in_specs = [pl.BlockSpec((blk, H), lambda i, o=h_row_blk: (o + i, 0))]
    args = [h]
    for o in p_offs:
        in_specs.append(pl.BlockSpec((2, blk, H), lambda i, o=o: (0, o + i, 0)))
        args.append(parts)
    for o in c_offs:
        in_specs.append(pl.BlockSpec((2, blk, H), lambda i, o=o: (0, o + i, 0)))
        args.append(cnts)
    for w in ws:
        in_specs.append(pl.BlockSpec((H, H), lambda i: (0, 0)))
        args.append(w)
    for a in (rw, rb.reshape(1, H), g.reshape(1, H), bt.reshape(1, H)):
        in_specs.append(pl.BlockSpec(a.shape, lambda i: (0,) * a.ndim))
        args.append(a)
    return pl.pallas_call(
        _make_update_body(nrel),
        grid=(N_EVENT // blk,),
        in_specs=in_specs,
        out_specs=pl.BlockSpec((blk, H), lambda i: (i, 0)),
        out_shape=jax.ShapeDtypeStruct((N_EVENT, H), jnp.float32),
    )(*args)


def _make_sc_agg(n_steps, lane, acc_rows):
    mesh = plsc.VectorSubcoreMesh(core_axis_name="c", subcore_axis_name="s",
                                  num_cores=NC, num_subcores=NS)
    per_tile = acc_rows // NS

    CH = CHUNK
    assert n_steps % CH == 0
    n_chunks = n_steps // CH

    @functools.partial(
        pl.kernel,
        out_type=jax.ShapeDtypeStruct((NC * acc_rows, H), jnp.float32),
        mesh=mesh,
        scratch_types=[
            pltpu.VMEM((2, CH, lane), jnp.int32),
            pltpu.VMEM((2, CH, lane), jnp.int32),
            [pltpu.VMEM((lane, H), jnp.float32) for _ in range(4)],
            [pltpu.SemaphoreType.DMA for _ in range(4)],
            [pltpu.SemaphoreType.DMA for _ in range(4)],
            [pltpu.SemaphoreType.DMA for _ in range(2)],
            pltpu.VMEM_SHARED((acc_rows, H), jnp.float32),
        ],
    )
    def agg(src_hbm, dst_hbm, table_hbm, zero_hbm, out_hbm,
            src_v, dst_v, bufs, gsems, ssems, isems, acc):
        cidx = lax.axis_index("c")
        sidx = lax.axis_index("s")
        wid = cidx * NS + sidx
        cbase = wid * n_chunks
        pltpu.sync_copy(src_hbm.at[cbase], src_v.at[0])
        pltpu.sync_copy(dst_hbm.at[cbase], dst_v.at[0])
        base = sidx * per_tile
        pltpu.sync_copy(zero_hbm, acc.at[pl.ds(base, per_tile)])
        plsc.subcore_barrier()

        def gather(j):
            cc, lj = divmod(j, CH)
            return pltpu.async_copy(table_hbm.at[src_v.at[cc % 2, lj]],
                                    bufs[j % 4], gsems[j % 4])

        gd = {0: gather(0), 1: gather(1)}
        sd = {}
        idx_d = None
        for j in range(n_steps):
            cc, lj = divmod(j, CH)
            if lj == 2 and cc + 1 < n_chunks:
                nxt = (cc + 1) % 2
                idx_d = (
                    pltpu.async_copy(src_hbm.at[cbase + cc + 1],
                                     src_v.at[nxt], isems[0]),
                    pltpu.async_copy(dst_hbm.at[cbase + cc + 1],
                                     dst_v.at[nxt], isems[1]),
                )
            if lj == CH - 2 and idx_d is not None:
                idx_d[0].wait()
                idx_d[1].wait()
                idx_d = None
            if j >= 2:
                sd.pop(j - 2).wait()
            if j + 2 < n_steps:
                gd[j + 2] = gather(j + 2)
            gd.pop(j).wait()
            sd[j] = pltpu.async_copy(bufs[j % 4],
                                     acc.at[dst_v.at[cc % 2, lj]],
                                     ssems[j % 4], add=True)
        sd.pop(n_steps - 2).wait()
        sd.pop(n_steps - 1).wait()
        plsc.subcore_barrier()
        pltpu.sync_copy(acc.at[pl.ds(base, per_tile)],
                        out_hbm.at[pl.ds(cidx * acc_rows + base, per_tile)])

    return agg


_make_sc_agg = functools.lru_cache(maxsize=None)(_make_sc_agg)


def _sc_aggA(*args):
    return _make_sc_agg(STEPSA, LANEA, ACCA_ROWS)(*args)


def _sc_aggB(*args):
    return _make_sc_agg(STEPSB, LANEB, ACCB_ROWS)(*args)


def _pad_edges(src, dst, n_steps, lane, dummy):
    tot = NC * NS * n_steps * lane
    pad = tot - dst.shape[0]
    spread = jnp.arange(pad, dtype=jnp.int32)
    shape = (NC * NS * (n_steps // CHUNK), CHUNK, lane)
    dst = jnp.concatenate([dst, dummy + spread % 64]).reshape(shape)
    src = jnp.concatenate([src, spread % N]).reshape(shape)
    return src, dst


def kernel(x_event, x_ioc, pW_e, pb_e, pg_e, pbt_e, pW_i, pb_i, pg_i, pbt_i,
           bases, comp, rootw, rootb, lng, lnb, ei0, ei1, ei2):
    h_e = _project(x_event, pW_e, pb_e, pg_e, pbt_e)
    h_i = _project(x_ioc, pW_i, pb_i, pg_i, pbt_i)
    h = jnp.concatenate([h_e, h_i], axis=0)

    sA = jnp.concatenate([ei1[0] + N_EVENT, ei2[0]])
    dA = jnp.concatenate([ei1[1], ei2[1] + N_EVENT])
    srcA, dstA = _pad_edges(sA, dA, STEPSA, LANEA, DUMMYA)
    srcB, dstB = _pad_edges(ei0[0], ei0[1], STEPSB, LANEB, DUMMYB)

    zerosA = jnp.zeros((ACCA_ROWS // NS, H), jnp.float32)
    zerosB = jnp.zeros((ACCB_ROWS // NS, H), jnp.float32)
    ones_tbl = jnp.ones((ONES_ROWS, H), jnp.float32)

    srcCA = (jnp.arange(srcA.size, dtype=jnp.int32) % ONES_ROWS
             ).reshape(srcA.shape)
    srcCB = (jnp.arange(srcB.size, dtype=jnp.int32) % ONES_ROWS
             ).reshape(srcB.shape)
    cntA = _sc_aggA(srcCA, dstA, ones_tbl, zerosA).reshape(NC, ACCA_ROWS, H)
    cntB = _sc_aggB(srcCB, dstB, ones_tbl, zerosB).reshape(NC, ACCB_ROWS, H)
    w_rel = _relation_weights(bases, comp)

    pa1 = _sc_aggA(srcA, dstA, h, zerosA).reshape(NC, ACCA_ROWS, H)
    pb1 = _sc_aggB(srcB, dstB, h, zerosB).reshape(NC, ACCB_ROWS, H)
    h1_e = _update(h, 0, pa1, cntA, p_offs=(0, 5), c_offs=(0, 5),
                   ws=(w_rel[0, 1], w_rel[0, 2]), rw=rootw[0], rb=rootb[0],
                   g=lng[0], bt=lnb[0])
    h1_i = _update(h, 5, pb1, cntB, p_offs=(0,), c_offs=(0,),
                   ws=(w_rel[0, 0],), rw=rootw[0], rb=rootb[0],
                   g=lng[0], bt=lnb[0])
    h1 = jnp.concatenate([h1_e, h1_i], axis=0)

    pa2 = _sc_aggA(srcA, dstA, h1, zerosA).reshape(NC, ACCA_ROWS, H)
    out = _update(h1, 0, pa2, cntA, p_offs=(0, 5), c_offs=(0, 5),
                  ws=(w_rel[1, 1], w_rel[1, 2]), rw=rootw[1], rb=rootb[1],
                  g=lng[1], bt=lnb[1])
    return out


# speedup vs baseline: 34.8349x; 1.1411x over previous
import functools

import jax
import jax.numpy as jnp
from jax import lax
from jax.experimental import pallas as pl
from jax.experimental.pallas import tpu as pltpu
from jax.experimental.pallas import tpu_sc as plsc

N_EVENT = 5000
N_IOC = 5000
N = N_EVENT + N_IOC
H = 128
NB = 8
R = 3
NC = 2
NS = 16
LANE = 128

ACCA_ROWS = 10112
DUMMYA = 10016
LANEA = 64
STEPSA = 100
CHUNK = 20
ACCB_ROWS = 5120
DUMMYB = 5024
LANEB = 64
STEPSB = 60
ONES_ROWS = 1024
STEPS2 = 50


def _proj_body(x_ref, w_ref, b_ref, g_ref, bt_ref, o_ref):
    y = jnp.dot(x_ref[...], w_ref[...], preferred_element_type=jnp.float32)
    y = y + b_ref[...]
    mu = jnp.mean(y, axis=-1, keepdims=True)
    var = jnp.mean((y - mu) ** 2, axis=-1, keepdims=True)
    y = (y - mu) * lax.rsqrt(var + 1e-5) * g_ref[...] + bt_ref[...]
    o_ref[...] = jnp.maximum(y, 0.0)


def _project(x, w, b, g, bt, blk=1000):
    n, d = x.shape
    return pl.pallas_call(
        _proj_body,
        grid=(n // blk,),
        in_specs=[
            pl.BlockSpec((blk, d), lambda i: (i, 0)),
            pl.BlockSpec((d, H), lambda i: (0, 0)),
            pl.BlockSpec((1, H), lambda i: (0, 0)),
            pl.BlockSpec((1, H), lambda i: (0, 0)),
            pl.BlockSpec((1, H), lambda i: (0, 0)),
        ],
        out_specs=pl.BlockSpec((blk, H), lambda i: (i, 0)),
        out_shape=jax.ShapeDtypeStruct((n, H), jnp.float32),
    )(x, w, b.reshape(1, H), g.reshape(1, H), bt.reshape(1, H))


def _relw_body(bases_ref, comp_ref, o_ref):
    l = pl.program_id(0)
    r = pl.program_id(1)
    acc = comp_ref[l, r, 0] * bases_ref[0, 0]
    for b in range(1, NB):
        acc = acc + comp_ref[l, r, b] * bases_ref[0, b]
    o_ref[0, 0] = acc


def _relation_weights(bases, comp):
    L = bases.shape[0]
    return pl.pallas_call(
        _relw_body,
        grid=(L, R),
        in_specs=[
            pl.BlockSpec((1, NB, H, H), lambda l, r: (l, 0, 0, 0)),
            pl.BlockSpec(memory_space=pltpu.SMEM),
        ],
        out_specs=pl.BlockSpec((1, 1, H, H), lambda l, r: (l, r, 0, 0)),
        out_shape=jax.ShapeDtypeStruct((L, R, H, H), jnp.float32),
    )(bases, comp)


def _make_update_body(nrel):
    def body(*refs):
        h_ref = refs[0]
        p_refs = refs[1:1 + nrel]
        c_refs = refs[1 + nrel:1 + 2 * nrel]
        w_refs = refs[1 + 2 * nrel:1 + 3 * nrel]
        rw_ref, rb_ref, g_ref, bt_ref, o_ref = refs[1 + 3 * nrel:]
        h = h_ref[...]
        out = jnp.dot(h, rw_ref[...], preferred_element_type=jnp.float32)
        out = out + rb_ref[...]
        for i in range(nrel):
            p = p_refs[i][...]
            a = p[0] + p[1]
            c = c_refs[i][...]
            cnt = c[0, :, 0:1] + c[1, :, 0:1]
            a = a * (1.0 / jnp.maximum(cnt, 1.0))
            out = out + jnp.dot(a, w_refs[i][...],
                                preferred_element_type=jnp.float32)
        mu = jnp.mean(out, axis=-1, keepdims=True)
        var = jnp.mean((out - mu) ** 2, axis=-1, keepdims=True)
        out = (out - mu) * lax.rsqrt(var + 1e-5) * g_ref[...] + bt_ref[...]
        o_ref[...] = jnp.maximum(out, 0.0) + h
    return body


def _update(h, h_row_blk, parts, cnts, p_offs, c_offs, ws, rw, rb, g, bt,
            blk=1000):
    nrel = len(p_offs)
    in_specs = [pl.BlockSpec((blk, H), lambda i, o=h_row_blk: (o + i, 0))]
    args = [h]
    for o in p_offs:
        in_specs.append(pl.BlockSpec((2, blk, H), lambda i, o=o: (0, o + i, 0)))
        args.append(parts)
    for o in c_offs:
        in_specs.append(pl.BlockSpec((2, blk, H), lambda i, o=o: (0, o + i, 0)))
        args.append(cnts)
    for w in ws:
        in_specs.append(pl.BlockSpec((H, H), lambda i: (0, 0)))
        args.append(w)
    for a in (rw, rb.reshape(1, H), g.reshape(1, H), bt.reshape(1, H)):
        in_specs.append(pl.BlockSpec(a.shape, lambda i: (0,) * a.ndim))
        args.append(a)
    return pl.pallas_call(
        _make_update_body(nrel),
        grid=(N_EVENT // blk,),
        in_specs=in_specs,
        out_specs=pl.BlockSpec((blk, H), lambda i: (i, 0)),
        out_shape=jax.ShapeDtypeStruct((N_EVENT, H), jnp.float32),
    )(*args)


def _make_sc_agg(n_steps, lane, acc_rows):
    mesh = plsc.VectorSubcoreMesh(core_axis_name="c", subcore_axis_name="s",
                                  num_cores=NC, num_subcores=NS)
    per_tile = acc_rows // NS

    CH = CHUNK
    assert n_steps % CH == 0
    n_chunks = n_steps // CH

    @functools.partial(
        pl.kernel,
        out_type=jax.ShapeDtypeStruct((NC * acc_rows, H), jnp.float32),
        mesh=mesh,
        scratch_types=[
            pltpu.VMEM((2, CH, lane), jnp.int32),
            pltpu.VMEM((2, CH, lane), jnp.int32),
            [pltpu.VMEM((lane, H), jnp.float32) for _ in range(4)],
            [pltpu.SemaphoreType.DMA for _ in range(4)],
            [pltpu.SemaphoreType.DMA for _ in range(4)],
            [pltpu.SemaphoreType.DMA for _ in range(2)],
            pltpu.VMEM_SHARED((acc_rows, H), jnp.float32),
        ],
    )
    def agg(src_hbm, dst_hbm, table_hbm, zero_hbm, out_hbm,
            src_v, dst_v, bufs, gsems, ssems, isems, acc):
        cidx = lax.axis_index("c")
        sidx = lax.axis_index("s")
        wid = cidx * NS + sidx
        cbase = wid * n_chunks
        pltpu.sync_copy(src_hbm.at[cbase], src_v.at[0])
        pltpu.sync_copy(dst_hbm.at[cbase], dst_v.at[0])
        base = sidx * per_tile
        pltpu.sync_copy(zero_hbm, acc.at[pl.ds(base, per_tile)])
        plsc.subcore_barrier()

        def gather(j):
            cc, lj = divmod(j, CH)
            return pltpu.async_copy(table_hbm.at[src_v.at[cc % 2, lj]],
                                    bufs[j % 4], gsems[j % 4])

        gd = {0: gather(0), 1: gather(1)}
        sd = {}
        idx_d = None
        for j in range(n_steps):
            cc, lj = divmod(j, CH)
            if lj == 2 and cc + 1 < n_chunks:
                nxt = (cc + 1) % 2
                idx_d = (
                    pltpu.async_copy(src_hbm.at[cbase + cc + 1],
                                     src_v.at[nxt], isems[0]),
                    pltpu.async_copy(dst_hbm.at[cbase + cc + 1],
                                     dst_v.at[nxt], isems[1]),
                )
            if lj == CH - 2 and idx_d is not None:
                idx_d[0].wait()
                idx_d[1].wait()
                idx_d = None
            if j >= 2:
                sd.pop(j - 2).wait()
            if j + 2 < n_steps:
                gd[j + 2] = gather(j + 2)
            gd.pop(j).wait()
            sd[j] = pltpu.async_copy(bufs[j % 4],
                                     acc.at[dst_v.at[cc % 2, lj]],
                                     ssems[j % 4], add=True)
        sd.pop(n_steps - 2).wait()
        sd.pop(n_steps - 1).wait()
        plsc.subcore_barrier()
        pltpu.sync_copy(acc.at[pl.ds(base, per_tile)],
                        out_hbm.at[pl.ds(cidx * acc_rows + base, per_tile)])

    return agg


def _make_sc_count(n_steps, lane, acc_rows):
    mesh = plsc.VectorSubcoreMesh(core_axis_name="c", subcore_axis_name="s",
                                  num_cores=NC, num_subcores=NS)
    per_tile = acc_rows // NS
    CH = 10
    assert n_steps % CH == 0
    n_chunks = n_steps // CH

    @functools.partial(
        pl.kernel,
        out_type=jax.ShapeDtypeStruct((NC * acc_rows, H), jnp.float32),
        mesh=mesh,
        scratch_types=[
            pltpu.VMEM((2, CH, lane), jnp.int32),
            pltpu.VMEM((lane, H), jnp.float32),
            [pltpu.SemaphoreType.DMA for _ in range(4)],
            pltpu.SemaphoreType.DMA,
            pltpu.VMEM_SHARED((acc_rows, H), jnp.float32),
        ],
    )
    def cnt(dst_hbm, ones_hbm, zero_hbm, out_hbm,
            dst_v, ones_v, ssems, isem, acc):
        cidx = lax.axis_index("c")
        sidx = lax.axis_index("s")
        wid = cidx * NS + sidx
        cbase = wid * n_chunks
        pltpu.sync_copy(dst_hbm.at[cbase], dst_v.at[0])
        pltpu.sync_copy(ones_hbm, ones_v)
        base = sidx * per_tile
        pltpu.sync_copy(zero_hbm, acc.at[pl.ds(base, per_tile)])
        plsc.subcore_barrier()

        sd = {}
        idx_d = None
        for j in range(n_steps):
            cc, lj = divmod(j, CH)
            if j >= 4:
                sd.pop(j - 4).wait()
            if lj == 4 and cc + 1 < n_chunks:
                idx_d = pltpu.async_copy(dst_hbm.at[cbase + cc + 1],
                                         dst_v.at[(cc + 1) % 2], isem)
            if lj == CH - 1 and idx_d is not None:
                idx_d.wait()
                idx_d = None
            sd[j] = pltpu.async_copy(ones_v, acc.at[dst_v.at[cc % 2, lj]],
                                     ssems[j % 4], add=True)
        for j in range(n_steps - 4, n_steps):
            sd.pop(j).wait()
        plsc.subcore_barrier()
        pltpu.sync_copy(acc.at[pl.ds(base, per_tile)],
                        out_hbm.at[pl.ds(cidx * acc_rows + base, per_tile)])

    return cnt


_make_sc_agg = functools.lru_cache(maxsize=None)(_make_sc_agg)
_make_sc_count = functools.lru_cache(maxsize=None)(_make_sc_count)


def _sc_aggA(*args):
    return _make_sc_agg(STEPSA, LANEA, ACCA_ROWS)(*args)


def _sc_aggB(*args):
    return _make_sc_agg(STEPSB, LANEB, ACCB_ROWS)(*args)


def _sc_cntA(*args):
    return _make_sc_count(STEPSA * LANEA // 128, 128, ACCA_ROWS)(*args)


def _sc_cntB(*args):
    return _make_sc_count(STEPSB * LANEB // 128, 128, ACCB_ROWS)(*args)


def _shape_idx(flat, n_steps, lane, ch):
    return flat.reshape(NC * NS * (n_steps // ch), ch, lane)


def _pad_edges(src, dst, n_steps, lane, dummy):
    tot = NC * NS * n_steps * lane
    pad = tot - dst.shape[0]
    spread = jnp.arange(pad, dtype=jnp.int32)
    dst = jnp.concatenate([dst, dummy + spread % 64])
    src = jnp.concatenate([src, spread % N])
    return src, dst


def kernel(x_event, x_ioc, pW_e, pb_e, pg_e, pbt_e, pW_i, pb_i, pg_i, pbt_i,
           bases, comp, rootw, rootb, lng, lnb, ei0, ei1, ei2):
    h_e = _project(x_event, pW_e, pb_e, pg_e, pbt_e)
    h_i = _project(x_ioc, pW_i, pb_i, pg_i, pbt_i)
    h = jnp.concatenate([h_e, h_i], axis=0)

    sA = jnp.concatenate([ei1[0] + N_EVENT, ei2[0]])
    dA = jnp.concatenate([ei1[1], ei2[1] + N_EVENT])
    sAf, dAf = _pad_edges(sA, dA, STEPSA, LANEA, DUMMYA)
    srcA = _shape_idx(sAf, STEPSA, LANEA, CHUNK)
    dstA = _shape_idx(dAf, STEPSA, LANEA, CHUNK)
    sBf, dBf = _pad_edges(ei0[0], ei0[1], STEPSB, LANEB, DUMMYB)
    srcB = _shape_idx(sBf, STEPSB, LANEB, CHUNK)
    dstB = _shape_idx(dBf, STEPSB, LANEB, CHUNK)

    zerosA = jnp.zeros((ACCA_ROWS // NS, H), jnp.float32)
    zerosB = jnp.zeros((ACCB_ROWS // NS, H), jnp.float32)
    ones128 = jnp.ones((128, H), jnp.float32)

    cntA = _sc_cntA(_shape_idx(dAf, STEPSA * LANEA // 128, 128, 10),
                    ones128, zerosA).reshape(NC, ACCA_ROWS, H)
    cntB = _sc_cntB(_shape_idx(dBf, STEPSB * LANEB // 128, 128, 10),
                    ones128, zerosB).reshape(NC, ACCB_ROWS, H)
    w_rel = _relation_weights(bases, comp)

    pa1 = _sc_aggA(srcA, dstA, h, zerosA).reshape(NC, ACCA_ROWS, H)
    pb1 = _sc_aggB(srcB, dstB, h, zerosB).reshape(NC, ACCB_ROWS, H)
    h1_e = _update(h, 0, pa1, cntA, p_offs=(0, 5), c_offs=(0, 5),
                   ws=(w_rel[0, 1], w_rel[0, 2]), rw=rootw[0], rb=rootb[0],
                   g=lng[0], bt=lnb[0])
    h1_i = _update(h, 5, pb1, cntB, p_offs=(0,), c_offs=(0,),
                   ws=(w_rel[0, 0],), rw=rootw[0], rb=rootb[0],
                   g=lng[0], bt=lnb[0])
    h1 = jnp.concatenate([h1_e, h1_i], axis=0)

    pa2 = _sc_aggA(srcA, dstA, h1, zerosA).reshape(NC, ACCA_ROWS, H)
    out = _update(h1, 0, pa2, cntA, p_offs=(0, 5), c_offs=(0, 5),
                  ws=(w_rel[1, 1], w_rel[1, 2]), rw=rootw[1], rb=rootb[1],
                  g=lng[1], bt=lnb[1])
    return out
